# baseline (speedup 1.0000x reference)
import functools

import jax
import jax.numpy as jnp
from jax import lax
from jax.experimental import pallas as pl
from jax.experimental.pallas import tpu as pltpu
from jax.experimental.pallas import tpu_sc as plsc

N = 10000
HID = 64
NC, NS, L = 2, 16, 16
NW = NC * NS
NPAD = 10240
CH = 128
ROWS_PER_TILE = NPAD // NS
NBUF = 8
GRP = 4


def _mesh():
    return plsc.VectorSubcoreMesh(core_axis_name="c", subcore_axis_name="s")


@functools.partial(jax.jit, static_argnums=(1,))
def _sc_hist(ei3, cap):
    @functools.partial(
        pl.kernel,
        out_type=jax.ShapeDtypeStruct((NW * NPAD,), jnp.float32),
        mesh=_mesh(),
        scratch_types=[
            pltpu.VMEM((NPAD,), jnp.float32),
            pltpu.VMEM((cap, CH), jnp.int32),
        ],
        compiler_params=pltpu.CompilerParams(needs_layout_passes=False),
    )
    def hist_k(ei_hbm, out_hbm, hist_v, idx_v):
        cid = lax.axis_index("c")
        sid = lax.axis_index("s")
        wid = cid * NS + sid

        def zbody(i, carry):
            hist_v[pl.ds(i * L, L)] = jnp.zeros((L,), jnp.float32)
            return carry

        lax.fori_loop(0, NPAD // L, zbody, 0)
        pltpu.sync_copy(ei_hbm.at[1, pl.ds(wid * cap, cap)], idx_v)

        ones = jnp.ones((L,), jnp.float32)

        def cbody(k, carry):
            for j in range(CH // L):
                idx = idx_v[k, pl.ds(j * L, L)]
                plsc.addupdate_scatter(hist_v, [idx], ones)
            return carry

        lax.fori_loop(0, cap, cbody, 0)
        pltpu.sync_copy(hist_v, out_hbm.at[pl.ds(wid * NPAD, NPAD)])

    return hist_k(ei3).reshape(NW, NPAD)


@functools.partial(jax.jit, static_argnums=(2,))
def _sc_scatter(u, ei3, cap):
    n_groups = cap // GRP

    @functools.partial(
        pl.kernel,
        out_type=jax.ShapeDtypeStruct((NC * NPAD, HID), jnp.float32),
        mesh=_mesh(),
        scratch_types=[
            pltpu.VMEM_SHARED((NPAD, HID), jnp.float32),
            pltpu.VMEM((cap, CH), jnp.int32),
            pltpu.VMEM((cap, CH), jnp.int32),
        ] + [pltpu.VMEM((CH, HID), jnp.float32) for _ in range(NBUF)] + [
            pltpu.SemaphoreType.DMA,
            pltpu.SemaphoreType.DMA,
        ],
        compiler_params=pltpu.CompilerParams(use_tc_tiling_on_sc=False),
    )
    def scat_k(u_hbm, ei_hbm, out_hbm, acc_sh, idx_s, idx_d, *rest):
        rows = rest[:NBUF]
        gsem, ssem = rest[NBUF], rest[NBUF + 1]
        cid = lax.axis_index("c")
        sid = lax.axis_index("s")
        wid = cid * NS + sid

        def zb(i, carry):
            rows[0][i, pl.ds(0, L)] = jnp.zeros((L,), jnp.float32)
            rows[0][i, pl.ds(L, L)] = jnp.zeros((L,), jnp.float32)
            rows[0][i, pl.ds(2 * L, L)] = jnp.zeros((L,), jnp.float32)
            rows[0][i, pl.ds(3 * L, L)] = jnp.zeros((L,), jnp.float32)
            return carry

        lax.fori_loop(0, CH, zb, 0)

        def zcopy(t, carry):
            pltpu.sync_copy(rows[0],
                            acc_sh.at[pl.ds(sid * ROWS_PER_TILE + t * CH, CH)])
            return carry

        lax.fori_loop(0, ROWS_PER_TILE // CH, zcopy, 0)

        pltpu.sync_copy(ei_hbm.at[0, pl.ds(wid * cap, cap)], idx_s)
        pltpu.sync_copy(ei_hbm.at[1, pl.ds(wid * cap, cap)], idx_d)
        plsc.subcore_barrier()

        def gather(k):
            pltpu.async_copy(u_hbm.at[idx_s.at[k]], rows[k % NBUF], gsem)

        def gather_wait(k):
            pltpu.make_async_copy(u_hbm.at[idx_s.at[k]], rows[k % NBUF],
                                  gsem).wait()

        def scat(k):
            pltpu.async_copy(rows[k % NBUF], acc_sh.at[idx_d.at[k]],
                             ssem, add=True)

        def scat_wait(k):
            pltpu.make_async_copy(rows[k % NBUF], acc_sh.at[idx_d.at[k]],
                                  ssem).wait()

        for j in range(GRP):
            gather(j)
        for g in range(n_groups):
            ks = [g * GRP + j for j in range(GRP)]
            for k in ks:
                gather_wait(k)
            for k in ks:
                if k + GRP < cap:
                    gather(k + GRP)
            for k in ks:
                scat(k)
            for k in ks:
                scat_wait(k)

        plsc.subcore_barrier()
        pltpu.sync_copy(
            acc_sh.at[pl.ds(sid * ROWS_PER_TILE, ROWS_PER_TILE)],
            out_hbm.at[pl.ds(cid * NPAD + sid * ROWS_PER_TILE,
                             ROWS_PER_TILE)])

    return scat_k(u, ei3)


def _tc_prep(x, W1, hist_t):
    def body(x_ref, w_ref, h_ref, u_ref, dinv_ref):
        deg = jnp.sum(h_ref[...], axis=1, keepdims=True) + 1.0
        dinv = lax.rsqrt(deg)[:N, :]
        h = jnp.dot(x_ref[...], w_ref[...],
                    preferred_element_type=jnp.float32)
        u_ref[...] = h * dinv
        dinv_ref[...] = dinv

    return pl.pallas_call(
        body,
        out_shape=[jax.ShapeDtypeStruct((N, HID), jnp.float32),
                   jax.ShapeDtypeStruct((N, 1), jnp.float32)],
    )(x, W1, hist_t)


def _tc_mid(p, u0, dinv, b1, W2):
    def body(p_ref, u_ref, dinv_ref, b_ref, w_ref, out_ref):
        dinv = dinv_ref[...]
        psum = p_ref[0:N, :] + p_ref[NPAD:NPAD + N, :]
        h0 = dinv * (psum + u_ref[...]) + b_ref[...]
        h0 = jnp.maximum(h0, 0.0)
        out_ref[...] = dinv * jnp.dot(h0, w_ref[...],
                                      preferred_element_type=jnp.float32)

    return pl.pallas_call(
        body,
        out_shape=jax.ShapeDtypeStruct((N, HID), jnp.float32),
    )(p, u0, dinv, b1, W2)


def _tc_final(p, u1, dinv, b2, W3, b3, W4, b4, W5, b5):
    def body(p_ref, u_ref, dinv_ref, b2_ref, w3_ref, b3_ref,
             w4_ref, b4_ref, w5_ref, b5_ref, out_ref):
        psum = p_ref[0:N, :] + p_ref[NPAD:NPAD + N, :]
        h1 = dinv_ref[...] * (psum + u_ref[...]) + b2_ref[...]
        mean = jnp.mean(h1, axis=0, keepdims=True)
        mx = jnp.max(h1, axis=0, keepdims=True)
        logits = jnp.sum(h1 * mean, axis=1, keepdims=True)
        m = jnp.max(logits, axis=0, keepdims=True)
        e = jnp.exp(logits - m)
        att = e / jnp.sum(e, axis=0, keepdims=True)
        attp = jnp.sum(h1 * att, axis=0, keepdims=True)
        comb = jnp.concatenate([mean, mx, attp], axis=1)
        g = jnp.maximum(
            jnp.dot(comb, w3_ref[...], preferred_element_type=jnp.float32)
            + b3_ref[...], 0.0)
        g = jnp.maximum(
            jnp.dot(g, w4_ref[...], preferred_element_type=jnp.float32)
            + b4_ref[...], 0.0)
        out_ref[...] = jnp.dot(g, w5_ref[...],
                               preferred_element_type=jnp.float32) + b5_ref[...]

    return pl.pallas_call(
        body,
        out_shape=jax.ShapeDtypeStruct((1, 128), jnp.float32),
    )(p, u1, dinv, b2, W3, b3, W4, b4, W5, b5)


def kernel(x, edge_index, W1, b1, W2, b2, W3, b3, W4, b4, W5, b5):
    E = edge_index.shape[1]
    tot_ch = E // CH
    cap = ((-(-tot_ch // NW)) + 7) // 8 * 8
    ei3 = edge_index.reshape(2, tot_ch, CH)
    n_pad_ch = NW * cap - tot_ch
    if n_pad_ch:
        pad_dst = N + (jnp.arange(n_pad_ch * CH, dtype=edge_index.dtype)
                       % (NPAD - N)).reshape(1, n_pad_ch, CH)
        pad_src = jnp.zeros((1, n_pad_ch, CH), edge_index.dtype)
        ei3 = jnp.concatenate(
            [ei3, jnp.concatenate([pad_src, pad_dst], axis=0)], axis=1)

    hist = _sc_hist(ei3, cap)
    u0, dinv = _tc_prep(x, W1, hist.T)
    p = _sc_scatter(u0, ei3, cap)
    u1 = _tc_mid(p, u0, dinv, b1.reshape(1, HID), W2)
    p2 = _sc_scatter(u1, ei3, cap)
    out = _tc_final(p2, u1, dinv, b2.reshape(1, HID),
                    W3, b3.reshape(1, -1), W4, b4.reshape(1, -1),
                    W5, b5.reshape(1, -1))
    return out


# speedup vs baseline: 2.2670x; 2.2670x over previous
import functools

import jax
import jax.numpy as jnp
from jax import lax
from jax.experimental import pallas as pl
from jax.experimental.pallas import tpu as pltpu
from jax.experimental.pallas import tpu_sc as plsc

N = 10000
HID = 64
NC, NS, L = 2, 16, 16
NW = NC * NS
NPAD = 10240
CH = 128
ROWS_PER_TILE = NPAD // NS
NBUF = 8
GRP = 4


def _mesh():
    return plsc.VectorSubcoreMesh(core_axis_name="c", subcore_axis_name="s")


@functools.partial(jax.jit, static_argnums=(1,))
def _sc_hist(ei3, cap):
    @functools.partial(
        pl.kernel,
        out_type=jax.ShapeDtypeStruct((NW * NPAD,), jnp.float32),
        mesh=_mesh(),
        scratch_types=[
            pltpu.VMEM((NPAD,), jnp.float32),
            pltpu.VMEM((cap, CH), jnp.int32),
        ],
        compiler_params=pltpu.CompilerParams(needs_layout_passes=False),
    )
    def hist_k(ei_hbm, out_hbm, hist_v, idx_v):
        cid = lax.axis_index("c")
        sid = lax.axis_index("s")
        wid = cid * NS + sid

        def zbody(i, carry):
            hist_v[pl.ds(i * L, L)] = jnp.zeros((L,), jnp.float32)
            return carry

        lax.fori_loop(0, NPAD // L, zbody, 0)
        pltpu.sync_copy(ei_hbm.at[1, pl.ds(wid * cap, cap)], idx_v)

        ones = jnp.ones((L,), jnp.float32)

        def cbody(k, carry):
            for j in range(CH // L):
                idx = idx_v[k, pl.ds(j * L, L)]
                plsc.addupdate_scatter(hist_v, [idx], ones)
            return carry

        lax.fori_loop(0, cap, cbody, 0)
        pltpu.sync_copy(hist_v, out_hbm.at[pl.ds(wid * NPAD, NPAD)])

    return hist_k(ei3).reshape(NW, NPAD)


@functools.partial(jax.jit, static_argnums=(2,))
def _sc_scatter(u, ei3, cap):
    n_groups = cap // GRP

    @functools.partial(
        pl.kernel,
        out_type=jax.ShapeDtypeStruct((NC * NPAD, HID), jnp.float32),
        mesh=_mesh(),
        scratch_types=[
            pltpu.VMEM_SHARED((NPAD, HID), jnp.float32),
            pltpu.VMEM((cap, CH), jnp.int32),
            pltpu.VMEM((cap, CH), jnp.int32),
        ] + [pltpu.VMEM((CH, HID), jnp.float32) for _ in range(NBUF)] + [
            pltpu.SemaphoreType.DMA,
            pltpu.SemaphoreType.DMA,
        ],
        compiler_params=pltpu.CompilerParams(use_tc_tiling_on_sc=False),
    )
    def scat_k(u_hbm, ei_hbm, out_hbm, acc_sh, idx_s, idx_d, *rest):
        rows = rest[:NBUF]
        gsem, ssem = rest[NBUF], rest[NBUF + 1]
        cid = lax.axis_index("c")
        sid = lax.axis_index("s")
        wid = cid * NS + sid

        def zb(i, carry):
            rows[0][i, pl.ds(0, L)] = jnp.zeros((L,), jnp.float32)
            rows[0][i, pl.ds(L, L)] = jnp.zeros((L,), jnp.float32)
            rows[0][i, pl.ds(2 * L, L)] = jnp.zeros((L,), jnp.float32)
            rows[0][i, pl.ds(3 * L, L)] = jnp.zeros((L,), jnp.float32)
            return carry

        lax.fori_loop(0, CH, zb, 0)

        def zcopy(t, carry):
            pltpu.sync_copy(rows[0],
                            acc_sh.at[pl.ds(sid * ROWS_PER_TILE + t * CH, CH)])
            return carry

        lax.fori_loop(0, ROWS_PER_TILE // CH, zcopy, 0)

        pltpu.sync_copy(ei_hbm.at[0, pl.ds(wid * cap, cap)], idx_s)
        pltpu.sync_copy(ei_hbm.at[1, pl.ds(wid * cap, cap)], idx_d)
        plsc.subcore_barrier()

        def gather(k):
            pltpu.async_copy(u_hbm.at[idx_s.at[k]], rows[k % NBUF], gsem)

        def gather_wait(k):
            pltpu.make_async_copy(u_hbm.at[idx_s.at[k]], rows[k % NBUF],
                                  gsem).wait()

        def scat(k):
            pltpu.async_copy(rows[k % NBUF], acc_sh.at[idx_d.at[k]],
                             ssem, add=True)

        def scat_wait(k):
            pltpu.make_async_copy(rows[k % NBUF], acc_sh.at[idx_d.at[k]],
                                  ssem).wait()

        for j in range(GRP):
            gather(j)
        for g in range(n_groups):
            ks = [g * GRP + j for j in range(GRP)]
            for k in ks:
                gather_wait(k)
            for k in ks:
                if k + GRP < cap:
                    gather(k + GRP)
            for k in ks:
                scat(k)
            for k in ks:
                scat_wait(k)

        plsc.subcore_barrier()
        pltpu.sync_copy(
            acc_sh.at[pl.ds(sid * ROWS_PER_TILE, ROWS_PER_TILE)],
            out_hbm.at[pl.ds(cid * NPAD + sid * ROWS_PER_TILE,
                             ROWS_PER_TILE)])

    return scat_k(u, ei3)


def _tc_prep(x, W1, hist_t):
    def body(x_ref, w_ref, h_ref, u_ref, dinv_ref):
        deg = jnp.sum(h_ref[...], axis=1, keepdims=True) + 1.0
        dinv = lax.rsqrt(deg)[:N, :]
        h = jnp.dot(x_ref[...], w_ref[...],
                    preferred_element_type=jnp.float32)
        u_ref[...] = h * dinv
        dinv_ref[...] = dinv

    return pl.pallas_call(
        body,
        out_shape=[jax.ShapeDtypeStruct((N, HID), jnp.float32),
                   jax.ShapeDtypeStruct((N, 1), jnp.float32)],
    )(x, W1, hist_t)


def _tc_mid(p, u0, dinv, b1, W2):
    def body(p_ref, u_ref, dinv_ref, b_ref, w_ref, out_ref):
        dinv = dinv_ref[...]
        psum = p_ref[0:N, :] + p_ref[NPAD:NPAD + N, :]
        h0 = dinv * (psum + u_ref[...]) + b_ref[...]
        h0 = jnp.maximum(h0, 0.0)
        out_ref[...] = dinv * jnp.dot(h0, w_ref[...],
                                      preferred_element_type=jnp.float32)

    return pl.pallas_call(
        body,
        out_shape=jax.ShapeDtypeStruct((N, HID), jnp.float32),
    )(p, u0, dinv, b1, W2)


def _tc_final(p, u1, dinv, b2, W3, b3, W4, b4, W5, b5):
    def body(p_ref, u_ref, dinv_ref, b2_ref, w3_ref, b3_ref,
             w4_ref, b4_ref, w5_ref, b5_ref, out_ref):
        psum = p_ref[0:N, :] + p_ref[NPAD:NPAD + N, :]
        h1 = dinv_ref[...] * (psum + u_ref[...]) + b2_ref[...]
        mean = jnp.mean(h1, axis=0, keepdims=True)
        mx = jnp.max(h1, axis=0, keepdims=True)
        logits = jnp.sum(h1 * mean, axis=1, keepdims=True)
        m = jnp.max(logits, axis=0, keepdims=True)
        e = jnp.exp(logits - m)
        att = e / jnp.sum(e, axis=0, keepdims=True)
        attp = jnp.sum(h1 * att, axis=0, keepdims=True)
        comb = jnp.concatenate([mean, mx, attp], axis=1)
        g = jnp.maximum(
            jnp.dot(comb, w3_ref[...], preferred_element_type=jnp.float32)
            + b3_ref[...], 0.0)
        g = jnp.maximum(
            jnp.dot(g, w4_ref[...], preferred_element_type=jnp.float32)
            + b4_ref[...], 0.0)
        out_ref[...] = jnp.dot(g, w5_ref[...],
                               preferred_element_type=jnp.float32) + b5_ref[...]

    return pl.pallas_call(
        body,
        out_shape=jax.ShapeDtypeStruct((1, 128), jnp.float32),
    )(p, u1, dinv, b2, W3, b3, W4, b4, W5, b5)


def kernel(x, edge_index, W1, b1, W2, b2, W3, b3, W4, b4, W5, b5):
    E = edge_index.shape[1]
    tot_ch = E // CH
    cap = ((-(-tot_ch // NW)) + 7) // 8 * 8
    ei3 = edge_index.reshape(2, tot_ch, CH)
    n_pad_ch = NW * cap - tot_ch
    if n_pad_ch:
        lin = jnp.arange(n_pad_ch * CH, dtype=edge_index.dtype)
        pad_dst = (N + lin % (NPAD - N)).reshape(1, n_pad_ch, CH)
        pad_src = ((lin * 79) % N).reshape(1, n_pad_ch, CH)
        ei3 = jnp.concatenate(
            [ei3, jnp.concatenate([pad_src, pad_dst], axis=0)], axis=1)

    hist = _sc_hist(ei3, cap)
    u0, dinv = _tc_prep(x, W1, hist.T)
    p = _sc_scatter(u0, ei3, cap)
    u1 = _tc_mid(p, u0, dinv, b1.reshape(1, HID), W2)
    p2 = _sc_scatter(u1, ei3, cap)
    out = _tc_final(p2, u1, dinv, b2.reshape(1, HID),
                    W3, b3.reshape(1, -1), W4, b4.reshape(1, -1),
                    W5, b5.reshape(1, -1))
    return out


# speedup vs baseline: 2.2974x; 1.0134x over previous
import functools

import jax
import jax.numpy as jnp
from jax import lax
from jax.experimental import pallas as pl
from jax.experimental.pallas import tpu as pltpu
from jax.experimental.pallas import tpu_sc as plsc

N = 10000
HID = 64
NC, NS, L = 2, 16, 16
NW = NC * NS
NPAD = 10240
CH = 128
ROWS_PER_TILE = NPAD // NS
NBUF = 8
GRP = 4


def _mesh():
    return plsc.VectorSubcoreMesh(core_axis_name="c", subcore_axis_name="s")


@functools.partial(jax.jit, static_argnums=(1,))
def _sc_hist(ei3, cap):
    @functools.partial(
        pl.kernel,
        out_type=jax.ShapeDtypeStruct((NW * NPAD,), jnp.float32),
        mesh=_mesh(),
        scratch_types=[
            pltpu.VMEM((NPAD,), jnp.float32),
            pltpu.VMEM((cap, CH), jnp.int32),
        ],
        compiler_params=pltpu.CompilerParams(needs_layout_passes=False),
    )
    def hist_k(ei_hbm, out_hbm, hist_v, idx_v):
        cid = lax.axis_index("c")
        sid = lax.axis_index("s")
        wid = cid * NS + sid

        def zbody(i, carry):
            hist_v[pl.ds(i * L, L)] = jnp.zeros((L,), jnp.float32)
            return carry

        lax.fori_loop(0, NPAD // L, zbody, 0)
        pltpu.sync_copy(ei_hbm.at[1, pl.ds(wid * cap, cap)], idx_v)

        ones = jnp.ones((L,), jnp.float32)

        def cbody(k, carry):
            for j in range(CH // L):
                idx = idx_v[k, pl.ds(j * L, L)]
                plsc.addupdate_scatter(hist_v, [idx], ones)
            return carry

        lax.fori_loop(0, cap, cbody, 0)
        pltpu.sync_copy(hist_v, out_hbm.at[pl.ds(wid * NPAD, NPAD)])

    return hist_k(ei3).reshape(NW, NPAD)


@functools.partial(jax.jit, static_argnums=(2,))
def _sc_scatter(u, ei3, cap):
    n_groups = cap // GRP

    @functools.partial(
        pl.kernel,
        out_type=jax.ShapeDtypeStruct((NC * NPAD, HID), jnp.float32),
        mesh=_mesh(),
        scratch_types=[
            pltpu.VMEM_SHARED((NPAD, HID), jnp.float32),
            pltpu.VMEM((cap, CH), jnp.int32),
            pltpu.VMEM((cap, CH), jnp.int32),
        ] + [pltpu.VMEM((CH, HID), jnp.float32) for _ in range(NBUF)] + [
            pltpu.SemaphoreType.DMA,
            pltpu.SemaphoreType.DMA,
        ],
        compiler_params=pltpu.CompilerParams(use_tc_tiling_on_sc=False),
    )
    def scat_k(u_hbm, ei_hbm, out_hbm, acc_sh, idx_s, idx_d, *rest):
        rows = rest[:NBUF]
        gsem, ssem = rest[NBUF], rest[NBUF + 1]
        cid = lax.axis_index("c")
        sid = lax.axis_index("s")
        wid = cid * NS + sid

        def zb(i, carry):
            rows[0][i, pl.ds(0, L)] = jnp.zeros((L,), jnp.float32)
            rows[0][i, pl.ds(L, L)] = jnp.zeros((L,), jnp.float32)
            rows[0][i, pl.ds(2 * L, L)] = jnp.zeros((L,), jnp.float32)
            rows[0][i, pl.ds(3 * L, L)] = jnp.zeros((L,), jnp.float32)
            return carry

        lax.fori_loop(0, CH, zb, 0)

        def zcopy(t, carry):
            pltpu.sync_copy(rows[0],
                            acc_sh.at[pl.ds(sid * ROWS_PER_TILE + t * CH, CH)])
            return carry

        lax.fori_loop(0, ROWS_PER_TILE // CH, zcopy, 0)

        pltpu.sync_copy(ei_hbm.at[0, pl.ds(wid * cap, cap)], idx_s)
        pltpu.sync_copy(ei_hbm.at[1, pl.ds(wid * cap, cap)], idx_d)
        plsc.subcore_barrier()

        def gather(k):
            pltpu.async_copy(u_hbm.at[idx_s.at[k]], rows[k % NBUF], gsem)

        def gather_wait(k):
            pltpu.make_async_copy(u_hbm.at[idx_s.at[k]], rows[k % NBUF],
                                  gsem).wait()

        def scat(k):
            pltpu.async_copy(rows[k % NBUF], acc_sh.at[idx_d.at[k]],
                             ssem, add=True)

        def scat_wait(k):
            pltpu.make_async_copy(rows[k % NBUF], acc_sh.at[idx_d.at[k]],
                                  ssem).wait()

        for j in range(GRP):
            gather(j)
        for g in range(n_groups):
            ks = [g * GRP + j for j in range(GRP)]
            for k in ks:
                gather_wait(k)
            for k in ks:
                if k + GRP < cap:
                    gather(k + GRP)
            for k in ks:
                scat(k)
            for k in ks:
                scat_wait(k)

        plsc.subcore_barrier()
        pltpu.sync_copy(
            acc_sh.at[pl.ds(sid * ROWS_PER_TILE, ROWS_PER_TILE)],
            out_hbm.at[pl.ds(cid * NPAD + sid * ROWS_PER_TILE,
                             ROWS_PER_TILE)])

    return scat_k(u, ei3)


def _tc_prep(x, W1, hist):
    def body(x_ref, w_ref, h_ref, u_ref, dinv_ref):
        deg = jnp.sum(h_ref[...], axis=0) + 1.0
        dinv = lax.rsqrt(deg)[:N].reshape(N, 1)
        h = jnp.dot(x_ref[...], w_ref[...],
                    preferred_element_type=jnp.float32)
        u_ref[...] = h * dinv
        dinv_ref[...] = dinv

    return pl.pallas_call(
        body,
        out_shape=[jax.ShapeDtypeStruct((N, HID), jnp.float32),
                   jax.ShapeDtypeStruct((N, 1), jnp.float32)],
    )(x, W1, hist)


def _tc_mid(p, u0, dinv, b1, W2):
    def body(p_ref, u_ref, dinv_ref, b_ref, w_ref, out_ref):
        dinv = dinv_ref[...]
        psum = p_ref[0:N, :] + p_ref[NPAD:NPAD + N, :]
        h0 = dinv * (psum + u_ref[...]) + b_ref[...]
        h0 = jnp.maximum(h0, 0.0)
        out_ref[...] = dinv * jnp.dot(h0, w_ref[...],
                                      preferred_element_type=jnp.float32)

    return pl.pallas_call(
        body,
        out_shape=jax.ShapeDtypeStruct((N, HID), jnp.float32),
    )(p, u0, dinv, b1, W2)


def _tc_final(p, u1, dinv, b2, W3, b3, W4, b4, W5, b5):
    def body(p_ref, u_ref, dinv_ref, b2_ref, w3_ref, b3_ref,
             w4_ref, b4_ref, w5_ref, b5_ref, out_ref):
        psum = p_ref[0:N, :] + p_ref[NPAD:NPAD + N, :]
        h1 = dinv_ref[...] * (psum + u_ref[...]) + b2_ref[...]
        mean = jnp.mean(h1, axis=0, keepdims=True)
        mx = jnp.max(h1, axis=0, keepdims=True)
        logits = jnp.sum(h1 * mean, axis=1, keepdims=True)
        m = jnp.max(logits, axis=0, keepdims=True)
        e = jnp.exp(logits - m)
        att = e / jnp.sum(e, axis=0, keepdims=True)
        attp = jnp.sum(h1 * att, axis=0, keepdims=True)
        comb = jnp.concatenate([mean, mx, attp], axis=1)
        g = jnp.maximum(
            jnp.dot(comb, w3_ref[...], preferred_element_type=jnp.float32)
            + b3_ref[...], 0.0)
        g = jnp.maximum(
            jnp.dot(g, w4_ref[...], preferred_element_type=jnp.float32)
            + b4_ref[...], 0.0)
        out_ref[...] = jnp.dot(g, w5_ref[...],
                               preferred_element_type=jnp.float32) + b5_ref[...]

    return pl.pallas_call(
        body,
        out_shape=jax.ShapeDtypeStruct((1, 128), jnp.float32),
    )(p, u1, dinv, b2, W3, b3, W4, b4, W5, b5)


def kernel(x, edge_index, W1, b1, W2, b2, W3, b3, W4, b4, W5, b5):
    E = edge_index.shape[1]
    tot_ch = E // CH
    align = 8 * GRP // (2 if GRP % 2 == 0 else 1)
    cap = -(-(-(-tot_ch // NW)) // align) * align
    ei3 = edge_index.reshape(2, tot_ch, CH)
    n_pad_ch = NW * cap - tot_ch
    if n_pad_ch:
        lin = jnp.arange(n_pad_ch * CH, dtype=edge_index.dtype)
        pad_dst = (N + lin % (NPAD - N)).reshape(1, n_pad_ch, CH)
        pad_src = ((lin * 79) % N).reshape(1, n_pad_ch, CH)
        ei3 = jnp.concatenate(
            [ei3, jnp.concatenate([pad_src, pad_dst], axis=0)], axis=1)

    hist = _sc_hist(ei3, cap)
    u0, dinv = _tc_prep(x, W1, hist)
    p = _sc_scatter(u0, ei3, cap)
    u1 = _tc_mid(p, u0, dinv, b1.reshape(1, HID), W2)
    p2 = _sc_scatter(u1, ei3, cap)
    out = _tc_final(p2, u1, dinv, b2.reshape(1, HID),
                    W3, b3.reshape(1, -1), W4, b4.reshape(1, -1),
                    W5, b5.reshape(1, -1))
    return out


# speedup vs baseline: 2.6230x; 1.1417x over previous
import functools

import jax
import jax.numpy as jnp
from jax import lax
from jax.experimental import pallas as pl
from jax.experimental.pallas import tpu as pltpu
from jax.experimental.pallas import tpu_sc as plsc

N = 10000
HID = 64
NC, NS, L = 2, 16, 16
NW = NC * NS
NPAD = 10240
CH = 128
ROWS_PER_TILE = NPAD // NS
NBUF = 8
GRP = 4


def _mesh():
    return plsc.VectorSubcoreMesh(core_axis_name="c", subcore_axis_name="s")


@functools.partial(jax.jit, static_argnums=(1,))
def _sc_hist(ei3, cap):
    @functools.partial(
        pl.kernel,
        out_type=jax.ShapeDtypeStruct((NW * NPAD,), jnp.float32),
        mesh=_mesh(),
        scratch_types=[
            pltpu.VMEM((NPAD,), jnp.float32),
            pltpu.VMEM((cap, CH), jnp.int32),
        ],
        compiler_params=pltpu.CompilerParams(needs_layout_passes=False),
    )
    def hist_k(ei_hbm, out_hbm, hist_v, idx_v):
        cid = lax.axis_index("c")
        sid = lax.axis_index("s")
        wid = cid * NS + sid

        def zbody(i, carry):
            hist_v[pl.ds(i * L, L)] = jnp.zeros((L,), jnp.float32)
            return carry

        lax.fori_loop(0, NPAD // L, zbody, 0)
        pltpu.sync_copy(ei_hbm.at[1, pl.ds(wid * cap, cap)], idx_v)

        ones = jnp.ones((L,), jnp.float32)

        def cbody(k, carry):
            for j in range(CH // L):
                idx = idx_v[k, pl.ds(j * L, L)]
                addr = (idx & 1) * (NPAD // 2) + (idx >> 1)
                plsc.addupdate_scatter(hist_v, [addr], ones)
            return carry

        lax.fori_loop(0, cap, cbody, 0)
        pltpu.sync_copy(hist_v, out_hbm.at[pl.ds(wid * NPAD, NPAD)])

    return hist_k(ei3).reshape(NW, NPAD)


@functools.partial(jax.jit, static_argnums=(2,))
def _sc_scatter(u, ei3, cap):
    n_groups = cap // GRP

    @functools.partial(
        pl.kernel,
        out_type=jax.ShapeDtypeStruct((NC * NPAD, HID), jnp.float32),
        mesh=_mesh(),
        scratch_types=[
            pltpu.VMEM_SHARED((NPAD, HID), jnp.float32),
            pltpu.VMEM((cap, CH), jnp.int32),
            pltpu.VMEM((cap, CH), jnp.int32),
        ] + [pltpu.VMEM((CH, HID), jnp.float32) for _ in range(NBUF)] + [
            pltpu.SemaphoreType.DMA,
            pltpu.SemaphoreType.DMA,
        ],
        compiler_params=pltpu.CompilerParams(use_tc_tiling_on_sc=False),
    )
    def scat_k(u_hbm, ei_hbm, out_hbm, acc_sh, idx_s, idx_d, *rest):
        rows = rest[:NBUF]
        gsem, ssem = rest[NBUF], rest[NBUF + 1]
        cid = lax.axis_index("c")
        sid = lax.axis_index("s")
        wid = cid * NS + sid

        def zb(i, carry):
            rows[0][i, pl.ds(0, L)] = jnp.zeros((L,), jnp.float32)
            rows[0][i, pl.ds(L, L)] = jnp.zeros((L,), jnp.float32)
            rows[0][i, pl.ds(2 * L, L)] = jnp.zeros((L,), jnp.float32)
            rows[0][i, pl.ds(3 * L, L)] = jnp.zeros((L,), jnp.float32)
            return carry

        lax.fori_loop(0, CH, zb, 0)

        def zcopy(t, carry):
            pltpu.sync_copy(rows[0],
                            acc_sh.at[pl.ds(sid * ROWS_PER_TILE + t * CH, CH)])
            return carry

        lax.fori_loop(0, ROWS_PER_TILE // CH, zcopy, 0)

        pltpu.sync_copy(ei_hbm.at[0, pl.ds(wid * cap, cap)], idx_s)
        pltpu.sync_copy(ei_hbm.at[1, pl.ds(wid * cap, cap)], idx_d)
        plsc.subcore_barrier()

        def gather(k):
            pltpu.async_copy(u_hbm.at[idx_s.at[k]], rows[k % NBUF], gsem)

        def gather_wait(k):
            pltpu.make_async_copy(u_hbm.at[idx_s.at[k]], rows[k % NBUF],
                                  gsem).wait()

        def scat(k):
            pltpu.async_copy(rows[k % NBUF], acc_sh.at[idx_d.at[k]],
                             ssem, add=True)

        def scat_wait(k):
            pltpu.make_async_copy(rows[k % NBUF], acc_sh.at[idx_d.at[k]],
                                  ssem).wait()

        for j in range(GRP):
            gather(j)
        for g in range(n_groups):
            ks = [g * GRP + j for j in range(GRP)]
            for k in ks:
                gather_wait(k)
            for k in ks:
                if k + GRP < cap:
                    gather(k + GRP)
            for k in ks:
                scat(k)
            for k in ks:
                scat_wait(k)

        plsc.subcore_barrier()
        pltpu.sync_copy(
            acc_sh.at[pl.ds(sid * ROWS_PER_TILE, ROWS_PER_TILE)],
            out_hbm.at[pl.ds(cid * NPAD + sid * ROWS_PER_TILE,
                             ROWS_PER_TILE)])

    return scat_k(u, ei3)


N2 = N // 2
PN2 = NC * NPAD // 2


def _tc_prep(xe, xo, W1, hist):
    def body(xe_ref, xo_ref, w_ref, h_ref, u_ref, dinvp_ref):
        dinv = lax.rsqrt(jnp.sum(h_ref[...], axis=0) + 1.0)
        dinv_e = dinv[0:N2].reshape(N2, 1)
        dinv_o = dinv[NPAD // 2:NPAD // 2 + N2].reshape(N2, 1)
        he = jnp.dot(xe_ref[...] * dinv_e, w_ref[...],
                     preferred_element_type=jnp.float32)
        ho = jnp.dot(xo_ref[...] * dinv_o, w_ref[...],
                     preferred_element_type=jnp.float32)
        u_ref[...] = jnp.concatenate([he, ho], axis=1)
        dinvp_ref[...] = jnp.concatenate(
            [jnp.broadcast_to(dinv_e, (N2, HID)),
             jnp.broadcast_to(dinv_o, (N2, HID))], axis=1)

    return pl.pallas_call(
        body,
        out_shape=[jax.ShapeDtypeStruct((N2, 2 * HID), jnp.float32),
                   jax.ShapeDtypeStruct((N2, 2 * HID), jnp.float32)],
    )(xe, xo, W1, hist)


def _tc_mid(p, u0, dinvp, b1p, W2p):
    def body(p_ref, u_ref, dinvp_ref, b_ref, w_ref, out_ref):
        dinvp = dinvp_ref[...]
        psum = p_ref[0:N2, :] + p_ref[NPAD // 2:NPAD // 2 + N2, :]
        h0 = dinvp * (psum + u_ref[...]) + b_ref[...]
        h0 = jnp.maximum(h0, 0.0)
        out_ref[...] = dinvp * jnp.dot(h0, w_ref[...],
                                       preferred_element_type=jnp.float32)

    return pl.pallas_call(
        body,
        out_shape=jax.ShapeDtypeStruct((N2, 2 * HID), jnp.float32),
    )(p, u0, dinvp, b1p, W2p)


def _tc_final(p, u1, dinvp, b2p, W3, b3, W4, b4, W5, b5):
    def body(p_ref, u_ref, dinvp_ref, b2_ref, w3_ref, b3_ref,
             w4_ref, b4_ref, w5_ref, b5_ref, out_ref):
        psum = p_ref[0:N2, :] + p_ref[NPAD // 2:NPAD // 2 + N2, :]
        h1 = dinvp_ref[...] * (psum + u_ref[...]) + b2_ref[...]
        m128 = jnp.mean(h1, axis=0, keepdims=True)
        mean = (m128[:, :HID] + m128[:, HID:]) * 0.5
        mx128 = jnp.max(h1, axis=0, keepdims=True)
        mx = jnp.maximum(mx128[:, :HID], mx128[:, HID:])
        mean2 = jnp.concatenate([mean, mean], axis=1)
        s = h1 * mean2
        le = jnp.sum(s[:, :HID], axis=1, keepdims=True)
        lo = jnp.sum(s[:, HID:], axis=1, keepdims=True)
        m = jnp.maximum(jnp.max(le, axis=0, keepdims=True),
                        jnp.max(lo, axis=0, keepdims=True))
        ee = jnp.exp(le - m)
        eo = jnp.exp(lo - m)
        z = jnp.sum(ee, axis=0, keepdims=True) + \
            jnp.sum(eo, axis=0, keepdims=True)
        attw = jnp.concatenate(
            [jnp.broadcast_to(ee / z, (N2, HID)),
             jnp.broadcast_to(eo / z, (N2, HID))], axis=1)
        ap128 = jnp.sum(h1 * attw, axis=0, keepdims=True)
        attp = ap128[:, :HID] + ap128[:, HID:]
        comb = jnp.concatenate([mean, mx, attp], axis=1)
        g = jnp.maximum(
            jnp.dot(comb, w3_ref[...], preferred_element_type=jnp.float32)
            + b3_ref[...], 0.0)
        g = jnp.maximum(
            jnp.dot(g, w4_ref[...], preferred_element_type=jnp.float32)
            + b4_ref[...], 0.0)
        out_ref[...] = jnp.dot(g, w5_ref[...],
                               preferred_element_type=jnp.float32) + b5_ref[...]

    return pl.pallas_call(
        body,
        out_shape=jax.ShapeDtypeStruct((1, 128), jnp.float32),
    )(p, u1, dinvp, b2p, W3, b3, W4, b4, W5, b5)


def kernel(x, edge_index, W1, b1, W2, b2, W3, b3, W4, b4, W5, b5):
    E = edge_index.shape[1]
    tot_ch = E // CH
    align = 8 * GRP // (2 if GRP % 2 == 0 else 1)
    cap = -(-(-(-tot_ch // NW)) // align) * align
    ei3 = edge_index.reshape(2, tot_ch, CH)
    n_pad_ch = NW * cap - tot_ch
    if n_pad_ch:
        lin = jnp.arange(n_pad_ch * CH, dtype=edge_index.dtype)
        pad_dst = (N + lin % (NPAD - N)).reshape(1, n_pad_ch, CH)
        pad_src = ((lin * 79) % N).reshape(1, n_pad_ch, CH)
        ei3 = jnp.concatenate(
            [ei3, jnp.concatenate([pad_src, pad_dst], axis=0)], axis=1)

    hist = _sc_hist(ei3, cap)
    u0p, dinvp = _tc_prep(x[0::2], x[1::2], W1, hist)
    p = _sc_scatter(u0p.reshape(N, HID), ei3, cap)
    W2p = jnp.zeros((2 * HID, 2 * HID), W2.dtype)
    W2p = W2p.at[:HID, :HID].set(W2).at[HID:, HID:].set(W2)
    b1p = jnp.tile(b1, 2).reshape(1, 2 * HID)
    u1p = _tc_mid(p.reshape(PN2, 2 * HID), u0p, dinvp, b1p, W2p)
    p2 = _sc_scatter(u1p.reshape(N, HID), ei3, cap)
    out = _tc_final(p2.reshape(PN2, 2 * HID), u1p, dinvp,
                    jnp.tile(b2, 2).reshape(1, 2 * HID),
                    W3, b3.reshape(1, -1), W4, b4.reshape(1, -1),
                    W5, b5.reshape(1, -1))
    return out
